# baseline (speedup 1.0000x reference)
import dataclasses
import functools

import jax
import jax.numpy as jnp
from jax import lax
from jax.experimental import pallas as pl
from jax.experimental.pallas import tpu as pltpu
from jax.experimental.pallas import tpu_sc as plsc

B, C, H, W = 2, 96, 384, 384
CP = 128
HW = H * W
NPIX = B * HW
NC, NS, L = 2, 16, 16
NW = NC * NS
ROWS_W = H // NW
P = 64
CPR = W // P
NCHUNK = ROWS_W * CPR
G = P // L
CB = C // L
HB = 8
WROWS = ROWS_W * W


def _floor(v):
  t = v.astype(jnp.int32)
  tf = t.astype(jnp.float32)
  adj = jnp.where(tf > v, 1, 0)
  return t - adj, tf - adj.astype(jnp.float32)


def _tc_transpose_in(x, b):
  def body(x_ref, o_ref):
    for h in range(HB):
      blk = x_ref[0, :, h, :]
      o_ref[0, h, :, 0:C] = blk.T
      o_ref[0, h, :, C:CP] = jnp.zeros((W, CP - C), jnp.float32)

  return pl.pallas_call(
      body,
      grid=(H // HB,),
      in_specs=[pl.BlockSpec((1, C, HB, W), lambda i: (b, 0, i, 0))],
      out_specs=pl.BlockSpec((1, HB, W, CP), lambda i: (0, i, 0, 0)),
      out_shape=jax.ShapeDtypeStruct((1, H, W, CP), jnp.float32),
  )(x)


def _tc_transpose_out(y4):
  def body(y_ref, o_ref):
    for h in range(HB):
      o_ref[0, :, h, :] = y_ref[0, h, :, 0:C].T

  return pl.pallas_call(
      body,
      grid=(B, H // HB),
      in_specs=[pl.BlockSpec((1, HB, W, CP), lambda b, i: (b, i, 0, 0))],
      out_specs=pl.BlockSpec((1, C, HB, W), lambda b, i: (b, 0, i, 0)),
      out_shape=jax.ShapeDtypeStruct((B, C, H, W), jnp.float32),
  )(y4)


def _sc_resample(xt, warp, b):
  mesh = plsc.VectorSubcoreMesh(core_axis_name="c", subcore_axis_name="s")
  cp = pltpu.CompilerParams()
  if "needs_layout_passes" in pltpu.CompilerParams.__dataclass_fields__:
    cp = dataclasses.replace(cp, needs_layout_passes=False)

  @functools.partial(
      pl.kernel,
      compiler_params=cp,
      out_type=jax.ShapeDtypeStruct((HW, CP), jnp.float32),
      mesh=mesh,
      scratch_types=[
          [[pltpu.VMEM((P,), jnp.int32) for _ in range(4)] for _ in range(2)],
          [[pltpu.VMEM((P,), jnp.float32) for _ in range(4)] for _ in range(2)],
          [[pltpu.VMEM((P, CP), jnp.float32) for _ in range(4)]
           for _ in range(2)],
          [pltpu.VMEM((WROWS,), jnp.float32) for _ in range(2)],
          [pltpu.VMEM((P, CP), jnp.float32) for _ in range(2)],
          [pltpu.SemaphoreType.DMA for _ in range(2)],
          [pltpu.SemaphoreType.DMA for _ in range(2)],
          pltpu.SemaphoreType.DMA,
      ],
  )
  def k(xt_hbm, warp_hbm, out_hbm, idx_vs, w_vs, row_vs, wp_vs, out_vs, sems,
        osems, wsem):
    c = lax.axis_index("c")
    s = lax.axis_index("s")
    wid = c * NS + s
    row0 = wid * ROWS_W
    q0 = row0 * W
    woff0 = 2 * b * HW
    woff1 = woff0 + HW

    cp0 = pltpu.async_copy(warp_hbm.at[pl.ds(woff0 + q0, WROWS)], wp_vs[0],
                           wsem)
    cp1 = pltpu.async_copy(warp_hbm.at[pl.ds(woff1 + q0, WROWS)], wp_vs[1],
                           wsem)
    cp0.wait()
    cp1.wait()

    def stage(t, st):
      r4 = t // CPR
      row = row0 + r4
      col0 = (t - r4 * CPR) * P
      loc = r4 * W + col0
      rowf = row.astype(jnp.float32)
      for g in range(G):
        colf = (col0 + g * L).astype(jnp.float32)
        ii = lax.iota(jnp.int32, L).astype(jnp.float32)
        sl = pl.ds(g * L, L)
        slw = pl.ds(loc + g * L, L)
        sx = colf + ii + wp_vs[0][slw]
        sy = rowf + wp_vs[1][slw]
        x0i, x0f = _floor(sx)
        y0i, y0f = _floor(sy)
        wx = sx - x0f
        wy = sy - y0f
        bx0 = (x0f >= 0.0) & (x0f <= W - 1.0)
        bx1 = (x0f >= -1.0) & (x0f <= W - 2.0)
        by0 = (y0f >= 0.0) & (y0f <= H - 1.0)
        by1 = (y0f >= -1.0) & (y0f <= H - 2.0)
        ix0 = jnp.clip(x0i, 0, W - 1)
        ix1 = jnp.clip(x0i + 1, 0, W - 1)
        ry0 = jnp.clip(y0i, 0, H - 1) * W
        ry1 = jnp.clip(y0i + 1, 0, H - 1) * W
        idx_vs[st][0][sl] = ry0 + ix0
        idx_vs[st][1][sl] = ry0 + ix1
        idx_vs[st][2][sl] = ry1 + ix0
        idx_vs[st][3][sl] = ry1 + ix1
        zero = jnp.zeros((L,), jnp.float32)
        w_vs[st][0][sl] = jnp.where(bx0 & by0, (1.0 - wx) * (1.0 - wy), zero)
        w_vs[st][1][sl] = jnp.where(bx1 & by0, wx * (1.0 - wy), zero)
        w_vs[st][2][sl] = jnp.where(bx0 & by1, (1.0 - wx) * wy, zero)
        w_vs[st][3][sl] = jnp.where(bx1 & by1, wx * wy, zero)
      for k4 in range(4):
        pltpu.async_copy(xt_hbm.at[idx_vs[st][k4]], row_vs[st][k4], sems[st])

    def drain(st):
      for k4 in range(4):
        pltpu.make_async_copy(xt_hbm.at[idx_vs[st][k4]], row_vs[st][k4],
                              sems[st]).wait()

    def out_dst(t):
      r4 = t // CPR
      return out_hbm.at[pl.ds(q0 + r4 * W + (t - r4 * CPR) * P, P)]

    def combine_out(t, st):
      @pl.when(t >= 2)
      def _():
        pltpu.make_async_copy(out_vs[st], out_dst(t), osems[st]).wait()

      @plsc.parallel_loop(0, P, 1, unroll=2)
      def _pix(pi):
        pidx = jnp.full((L,), pi, jnp.int32)
        ws = [plsc.load_gather(w_vs[st][k4], [pidx]) for k4 in range(4)]
        for cb in range(CB):
          sl = pl.ds(cb * L, L)
          acc = ws[0] * row_vs[st][0][pi, sl]
          acc = acc + ws[1] * row_vs[st][1][pi, sl]
          acc = acc + ws[2] * row_vs[st][2][pi, sl]
          acc = acc + ws[3] * row_vs[st][3][pi, sl]
          out_vs[st][pi, sl] = acc

      pltpu.async_copy(out_vs[st], out_dst(t), osems[st])

    stage(jnp.int32(0), 0)

    @pl.loop(0, NCHUNK, step=2)
    def _chunks(t):
      stage(t + 1, 1)
      drain(0)
      combine_out(t, 0)

      @pl.when(t + 2 < NCHUNK)
      def _():
        stage(t + 2, 0)

      drain(1)
      combine_out(t + 1, 1)

    pltpu.make_async_copy(out_vs[0], out_dst(jnp.int32(NCHUNK - 2)),
                          osems[0]).wait()
    pltpu.make_async_copy(out_vs[1], out_dst(jnp.int32(NCHUNK - 1)),
                          osems[1]).wait()

  return k(xt, warp)


def kernel(x, warp):
  xf = x.astype(jnp.float32)
  wf = warp.astype(jnp.float32).reshape(2 * NPIX)
  xt0 = _tc_transpose_in(xf, 0)
  out0 = _sc_resample(xt0.reshape(HW, CP), wf, 0)
  xt1 = _tc_transpose_in(xf, 1)
  out1 = _sc_resample(xt1.reshape(HW, CP), wf, 1)
  y4 = jnp.stack([out0.reshape(H, W, CP), out1.reshape(H, W, CP)])
  return _tc_transpose_out(y4)


# speedup vs baseline: 1.2425x; 1.2425x over previous
import dataclasses
import functools

import jax
import jax.numpy as jnp
from jax import lax
from jax.experimental import pallas as pl
from jax.experimental.pallas import tpu as pltpu
from jax.experimental.pallas import tpu_sc as plsc

B, C, H, W = 2, 96, 384, 384
CP = 128
HW = H * W
NPIX = B * HW
NC, NS, L = 2, 16, 16
NW = NC * NS
ROWS_W = H // NW
P = 64
CPR = W // P
NCHUNK = ROWS_W * CPR
G = P // L
CB = C // L
HB = 8
WROWS = ROWS_W * W


def _floor(v):
  t = v.astype(jnp.int32)
  tf = t.astype(jnp.float32)
  adj = jnp.where(tf > v, 1, 0)
  return t - adj, tf - adj.astype(jnp.float32)


def _tc_transpose_in(x, b):
  def body(x_ref, o_ref):
    for h in range(HB):
      blk = x_ref[0, :, h, :]
      o_ref[0, h, :, 0:C] = blk.T
      o_ref[0, h, :, C:CP] = jnp.zeros((W, CP - C), jnp.float32)

  return pl.pallas_call(
      body,
      grid=(H // HB,),
      in_specs=[pl.BlockSpec((1, C, HB, W), lambda i: (b, 0, i, 0))],
      out_specs=pl.BlockSpec((1, HB, W, CP), lambda i: (0, i, 0, 0)),
      out_shape=jax.ShapeDtypeStruct((1, H, W, CP), jnp.float32),
  )(x)


def _tc_transpose_out_first(y1):
  def body(y_ref, o_ref):
    for h in range(HB):
      o_ref[0, :, h, :] = y_ref[0, h, :, 0:C].T

  return pl.pallas_call(
      body,
      grid=(H // HB,),
      in_specs=[pl.BlockSpec((1, HB, W, CP), lambda i: (0, i, 0, 0))],
      out_specs=pl.BlockSpec((1, C, HB, W), lambda i: (0, 0, i, 0)),
      out_shape=jax.ShapeDtypeStruct((B, C, H, W), jnp.float32),
  )(y1)


def _tc_transpose_out_second(y_acc, y1):
  def body(y_acc_ref, y_ref, o_ref):
    del y_acc_ref
    for h in range(HB):
      o_ref[0, :, h, :] = y_ref[0, h, :, 0:C].T

  return pl.pallas_call(
      body,
      grid=(H // HB,),
      in_specs=[
          pl.BlockSpec(memory_space=pl.ANY),
          pl.BlockSpec((1, HB, W, CP), lambda i: (0, i, 0, 0)),
      ],
      out_specs=pl.BlockSpec((1, C, HB, W), lambda i: (1, 0, i, 0)),
      out_shape=jax.ShapeDtypeStruct((B, C, H, W), jnp.float32),
      input_output_aliases={0: 0},
  )(y_acc, y1)


def _sc_resample(xt, warp, b):
  mesh = plsc.VectorSubcoreMesh(core_axis_name="c", subcore_axis_name="s")
  cp = pltpu.CompilerParams()
  if "needs_layout_passes" in pltpu.CompilerParams.__dataclass_fields__:
    cp = dataclasses.replace(cp, needs_layout_passes=False)

  @functools.partial(
      pl.kernel,
      compiler_params=cp,
      out_type=jax.ShapeDtypeStruct((HW, CP), jnp.float32),
      mesh=mesh,
      scratch_types=[
          [[pltpu.VMEM((P,), jnp.int32) for _ in range(4)] for _ in range(2)],
          [[pltpu.VMEM((P,), jnp.float32) for _ in range(4)] for _ in range(2)],
          [[pltpu.VMEM((P, CP), jnp.float32) for _ in range(4)]
           for _ in range(2)],
          [pltpu.VMEM((WROWS,), jnp.float32) for _ in range(2)],
          [pltpu.VMEM((P, CP), jnp.float32) for _ in range(2)],
          [pltpu.SemaphoreType.DMA for _ in range(2)],
          [pltpu.SemaphoreType.DMA for _ in range(2)],
          pltpu.SemaphoreType.DMA,
      ],
  )
  def k(xt_hbm, warp_hbm, out_hbm, idx_vs, w_vs, row_vs, wp_vs, out_vs, sems,
        osems, wsem):
    c = lax.axis_index("c")
    s = lax.axis_index("s")
    wid = c * NS + s
    row0 = wid * ROWS_W
    q0 = row0 * W
    woff0 = 2 * b * HW
    woff1 = woff0 + HW

    cp0 = pltpu.async_copy(warp_hbm.at[pl.ds(woff0 + q0, WROWS)], wp_vs[0],
                           wsem)
    cp1 = pltpu.async_copy(warp_hbm.at[pl.ds(woff1 + q0, WROWS)], wp_vs[1],
                           wsem)
    cp0.wait()
    cp1.wait()

    def stage(t, st):
      r4 = t // CPR
      row = row0 + r4
      col0 = (t - r4 * CPR) * P
      loc = r4 * W + col0
      rowf = row.astype(jnp.float32)
      for g in range(G):
        colf = (col0 + g * L).astype(jnp.float32)
        ii = lax.iota(jnp.int32, L).astype(jnp.float32)
        sl = pl.ds(g * L, L)
        slw = pl.ds(loc + g * L, L)
        sx = colf + ii + wp_vs[0][slw]
        sy = rowf + wp_vs[1][slw]
        x0i, x0f = _floor(sx)
        y0i, y0f = _floor(sy)
        wx = sx - x0f
        wy = sy - y0f
        bx0 = (x0f >= 0.0) & (x0f <= W - 1.0)
        bx1 = (x0f >= -1.0) & (x0f <= W - 2.0)
        by0 = (y0f >= 0.0) & (y0f <= H - 1.0)
        by1 = (y0f >= -1.0) & (y0f <= H - 2.0)
        ix0 = jnp.clip(x0i, 0, W - 1)
        ix1 = jnp.clip(x0i + 1, 0, W - 1)
        ry0 = jnp.clip(y0i, 0, H - 1) * W
        ry1 = jnp.clip(y0i + 1, 0, H - 1) * W
        idx_vs[st][0][sl] = ry0 + ix0
        idx_vs[st][1][sl] = ry0 + ix1
        idx_vs[st][2][sl] = ry1 + ix0
        idx_vs[st][3][sl] = ry1 + ix1
        zero = jnp.zeros((L,), jnp.float32)
        w_vs[st][0][sl] = jnp.where(bx0 & by0, (1.0 - wx) * (1.0 - wy), zero)
        w_vs[st][1][sl] = jnp.where(bx1 & by0, wx * (1.0 - wy), zero)
        w_vs[st][2][sl] = jnp.where(bx0 & by1, (1.0 - wx) * wy, zero)
        w_vs[st][3][sl] = jnp.where(bx1 & by1, wx * wy, zero)
      for k4 in range(4):
        pltpu.async_copy(xt_hbm.at[idx_vs[st][k4]], row_vs[st][k4], sems[st])

    def drain(st):
      for k4 in range(4):
        pltpu.make_async_copy(xt_hbm.at[idx_vs[st][k4]], row_vs[st][k4],
                              sems[st]).wait()

    def out_dst(t):
      r4 = t // CPR
      return out_hbm.at[pl.ds(q0 + r4 * W + (t - r4 * CPR) * P, P)]

    def combine_out(t, st):
      @pl.when(t >= 2)
      def _():
        pltpu.make_async_copy(out_vs[st], out_dst(t), osems[st]).wait()

      @plsc.parallel_loop(0, P, 1, unroll=2)
      def _pix(pi):
        pidx = jnp.full((L,), pi, jnp.int32)
        ws = [plsc.load_gather(w_vs[st][k4], [pidx]) for k4 in range(4)]
        for cb in range(CB):
          sl = pl.ds(cb * L, L)
          acc = ws[0] * row_vs[st][0][pi, sl]
          acc = acc + ws[1] * row_vs[st][1][pi, sl]
          acc = acc + ws[2] * row_vs[st][2][pi, sl]
          acc = acc + ws[3] * row_vs[st][3][pi, sl]
          out_vs[st][pi, sl] = acc

      pltpu.async_copy(out_vs[st], out_dst(t), osems[st])

    stage(jnp.int32(0), 0)

    @pl.loop(0, NCHUNK, step=2)
    def _chunks(t):
      stage(t + 1, 1)
      drain(0)
      combine_out(t, 0)

      @pl.when(t + 2 < NCHUNK)
      def _():
        stage(t + 2, 0)

      drain(1)
      combine_out(t + 1, 1)

    pltpu.make_async_copy(out_vs[0], out_dst(jnp.int32(NCHUNK - 2)),
                          osems[0]).wait()
    pltpu.make_async_copy(out_vs[1], out_dst(jnp.int32(NCHUNK - 1)),
                          osems[1]).wait()

  return k(xt, warp)


def kernel(x, warp):
  xf = x.astype(jnp.float32)
  wf = warp.astype(jnp.float32).reshape(2 * NPIX)
  xt0 = _tc_transpose_in(xf, 0)
  out0 = _sc_resample(xt0.reshape(HW, CP), wf, 0)
  xt1 = _tc_transpose_in(xf, 1)
  out1 = _sc_resample(xt1.reshape(HW, CP), wf, 1)
  y_acc = _tc_transpose_out_first(out0.reshape(1, H, W, CP))
  return _tc_transpose_out_second(y_acc, out1.reshape(1, H, W, CP))


# speedup vs baseline: 1.2484x; 1.0047x over previous
import dataclasses
import functools

import jax
import jax.numpy as jnp
from jax import lax
from jax.experimental import pallas as pl
from jax.experimental.pallas import tpu as pltpu
from jax.experimental.pallas import tpu_sc as plsc

B, C, H, W = 2, 96, 384, 384
CP = 128
HW = H * W
NPIX = B * HW
NC, NS, L = 2, 16, 16
NW = NC * NS
ROWS_W = H // NW
P = 96
CPR = W // P
NCHUNK = ROWS_W * CPR
G = P // L
CB = C // L
HB = 8
SLAB_ROWS = 4
SLAB_PX = SLAB_ROWS * W
CHUNKS_PER_SLAB = SLAB_ROWS * CPR


def _floor(v):
  t = v.astype(jnp.int32)
  tf = t.astype(jnp.float32)
  adj = jnp.where(tf > v, 1, 0)
  return t - adj, tf - adj.astype(jnp.float32)


def _tc_transpose_in(x, b):
  def body(x_ref, o_ref):
    for h in range(HB):
      blk = x_ref[0, :, h, :]
      o_ref[0, h, :, 0:C] = blk.T
      o_ref[0, h, :, C:CP] = jnp.zeros((W, CP - C), jnp.float32)

  return pl.pallas_call(
      body,
      grid=(H // HB,),
      in_specs=[pl.BlockSpec((1, C, HB, W), lambda i: (b, 0, i, 0))],
      out_specs=pl.BlockSpec((1, HB, W, CP), lambda i: (0, i, 0, 0)),
      out_shape=jax.ShapeDtypeStruct((1, H, W, CP), jnp.float32),
  )(x)


def _tc_transpose_out_first(y1):
  def body(y_ref, o_ref):
    for h in range(HB):
      o_ref[0, :, h, :] = y_ref[0, h, :, 0:C].T

  return pl.pallas_call(
      body,
      grid=(H // HB,),
      in_specs=[pl.BlockSpec((1, HB, W, CP), lambda i: (0, i, 0, 0))],
      out_specs=pl.BlockSpec((1, C, HB, W), lambda i: (0, 0, i, 0)),
      out_shape=jax.ShapeDtypeStruct((B, C, H, W), jnp.float32),
  )(y1)


def _tc_transpose_out_second(y_acc, y1):
  def body(y_acc_ref, y_ref, o_ref):
    del y_acc_ref
    for h in range(HB):
      o_ref[0, :, h, :] = y_ref[0, h, :, 0:C].T

  return pl.pallas_call(
      body,
      grid=(H // HB,),
      in_specs=[
          pl.BlockSpec(memory_space=pl.ANY),
          pl.BlockSpec((1, HB, W, CP), lambda i: (0, i, 0, 0)),
      ],
      out_specs=pl.BlockSpec((1, C, HB, W), lambda i: (1, 0, i, 0)),
      out_shape=jax.ShapeDtypeStruct((B, C, H, W), jnp.float32),
      input_output_aliases={0: 0},
  )(y_acc, y1)


def _sc_resample(xt, warp, b):
  mesh = plsc.VectorSubcoreMesh(core_axis_name="c", subcore_axis_name="s")
  cp = pltpu.CompilerParams()
  if "needs_layout_passes" in pltpu.CompilerParams.__dataclass_fields__:
    cp = dataclasses.replace(cp, needs_layout_passes=False)

  @functools.partial(
      pl.kernel,
      compiler_params=cp,
      out_type=jax.ShapeDtypeStruct((HW, CP), jnp.float32),
      mesh=mesh,
      scratch_types=[
          [[pltpu.VMEM((P,), jnp.int32) for _ in range(4)] for _ in range(2)],
          [[pltpu.VMEM((P,), jnp.float32) for _ in range(4)] for _ in range(2)],
          [[pltpu.VMEM((P, CP), jnp.float32) for _ in range(4)]
           for _ in range(2)],
          [pltpu.VMEM((SLAB_PX,), jnp.float32) for _ in range(2)],
          [pltpu.VMEM((P, CP), jnp.float32) for _ in range(2)],
          [pltpu.SemaphoreType.DMA for _ in range(2)],
          [pltpu.SemaphoreType.DMA for _ in range(2)],
          pltpu.SemaphoreType.DMA,
      ],
  )
  def k(xt_hbm, warp_hbm, out_hbm, idx_vs, w_vs, row_vs, wp_vs, out_vs, sems,
        osems, wsem):
    c = lax.axis_index("c")
    s = lax.axis_index("s")
    wid = c * NS + s
    row0 = wid * ROWS_W
    q0 = row0 * W
    woff0 = 2 * b * HW
    woff1 = woff0 + HW

    def load_slab(sb):
      q = q0 + sb * SLAB_PX
      cp0 = pltpu.async_copy(warp_hbm.at[pl.ds(woff0 + q, SLAB_PX)], wp_vs[0],
                             wsem)
      cp1 = pltpu.async_copy(warp_hbm.at[pl.ds(woff1 + q, SLAB_PX)], wp_vs[1],
                             wsem)
      cp0.wait()
      cp1.wait()

    load_slab(jnp.int32(0))

    def stage(t, st):
      r4 = t // CPR
      row = row0 + r4
      col0 = (t - r4 * CPR) * P
      loc = (r4 % SLAB_ROWS) * W + col0
      rowf = row.astype(jnp.float32)
      for g in range(G):
        colf = (col0 + g * L).astype(jnp.float32)
        ii = lax.iota(jnp.int32, L).astype(jnp.float32)
        sl = pl.ds(g * L, L)
        slw = pl.ds(loc + g * L, L)
        sx = colf + ii + wp_vs[0][slw]
        sy = rowf + wp_vs[1][slw]
        x0i, x0f = _floor(sx)
        y0i, y0f = _floor(sy)
        wx = sx - x0f
        wy = sy - y0f
        bx0 = (x0f >= 0.0) & (x0f <= W - 1.0)
        bx1 = (x0f >= -1.0) & (x0f <= W - 2.0)
        by0 = (y0f >= 0.0) & (y0f <= H - 1.0)
        by1 = (y0f >= -1.0) & (y0f <= H - 2.0)
        ix0 = jnp.clip(x0i, 0, W - 1)
        ix1 = jnp.clip(x0i + 1, 0, W - 1)
        ry0 = jnp.clip(y0i, 0, H - 1) * W
        ry1 = jnp.clip(y0i + 1, 0, H - 1) * W
        idx_vs[st][0][sl] = ry0 + ix0
        idx_vs[st][1][sl] = ry0 + ix1
        idx_vs[st][2][sl] = ry1 + ix0
        idx_vs[st][3][sl] = ry1 + ix1
        zero = jnp.zeros((L,), jnp.float32)
        w_vs[st][0][sl] = jnp.where(bx0 & by0, (1.0 - wx) * (1.0 - wy), zero)
        w_vs[st][1][sl] = jnp.where(bx1 & by0, wx * (1.0 - wy), zero)
        w_vs[st][2][sl] = jnp.where(bx0 & by1, (1.0 - wx) * wy, zero)
        w_vs[st][3][sl] = jnp.where(bx1 & by1, wx * wy, zero)
      for k4 in range(4):
        pltpu.async_copy(xt_hbm.at[idx_vs[st][k4]], row_vs[st][k4], sems[st])

    def drain(st):
      for k4 in range(4):
        pltpu.make_async_copy(xt_hbm.at[idx_vs[st][k4]], row_vs[st][k4],
                              sems[st]).wait()

    def out_dst(t):
      r4 = t // CPR
      return out_hbm.at[pl.ds(q0 + r4 * W + (t - r4 * CPR) * P, P)]

    def combine_out(t, st):
      @pl.when(t >= 2)
      def _():
        pltpu.make_async_copy(out_vs[st], out_dst(t), osems[st]).wait()

      @plsc.parallel_loop(0, P, 1, unroll=4)
      def _pix(pi):
        pidx = jnp.full((L,), pi, jnp.int32)
        ws = [plsc.load_gather(w_vs[st][k4], [pidx]) for k4 in range(4)]
        for cb in range(CB):
          sl = pl.ds(cb * L, L)
          acc = ws[0] * row_vs[st][0][pi, sl]
          acc = acc + ws[1] * row_vs[st][1][pi, sl]
          acc = acc + ws[2] * row_vs[st][2][pi, sl]
          acc = acc + ws[3] * row_vs[st][3][pi, sl]
          out_vs[st][pi, sl] = acc

      pltpu.async_copy(out_vs[st], out_dst(t), osems[st])

    stage(jnp.int32(0), 0)

    @pl.loop(0, NCHUNK, step=2)
    def _chunks(t):
      stage(t + 1, 1)
      drain(0)
      combine_out(t, 0)

      @pl.when(t + 2 < NCHUNK)
      def _():
        @pl.when((t + 2) % CHUNKS_PER_SLAB == 0)
        def _():
          load_slab((t + 2) // CHUNKS_PER_SLAB)

        stage(t + 2, 0)

      drain(1)
      combine_out(t + 1, 1)

    pltpu.make_async_copy(out_vs[0], out_dst(jnp.int32(NCHUNK - 2)),
                          osems[0]).wait()
    pltpu.make_async_copy(out_vs[1], out_dst(jnp.int32(NCHUNK - 1)),
                          osems[1]).wait()

  return k(xt, warp)


def kernel(x, warp):
  xf = x.astype(jnp.float32)
  wf = warp.astype(jnp.float32).reshape(2 * NPIX)
  xt0 = _tc_transpose_in(xf, 0)
  out0 = _sc_resample(xt0.reshape(HW, CP), wf, 0)
  xt1 = _tc_transpose_in(xf, 1)
  out1 = _sc_resample(xt1.reshape(HW, CP), wf, 1)
  y_acc = _tc_transpose_out_first(out0.reshape(1, H, W, CP))
  return _tc_transpose_out_second(y_acc, out1.reshape(1, H, W, CP))
